# Initial kernel scaffold; baseline (speedup 1.0000x reference)
import jax
import jax.numpy as jnp
from jax.experimental import pallas as pl


def kernel(item_scores, feat_scores, cand_item):
    raise NotImplementedError("write your pallas kernel here")


# speedup vs baseline: 1.3006x; 1.3006x over previous
import jax
import jax.numpy as jnp
from jax.experimental import pallas as pl
from jax.experimental.pallas import tpu as pltpu

B = 128
V = 100000
N_FEAT = 25
TOPK = 10

ROWS_PER_BLOCK = 8
V_PAD = 100096


def _topk_softmax_kernel(item_ref, feat_ref, out_ref):
    x = item_ref[...]
    iota = jax.lax.broadcasted_iota(jnp.int32, x.shape, 1)
    neg_inf = jnp.float32(-jnp.inf)
    vals = []
    for _ in range(TOPK):
        m = jnp.max(x, axis=1, keepdims=True)
        eq = x == m
        first = jnp.min(jnp.where(eq, iota, jnp.int32(V_PAD)), axis=1,
                        keepdims=True)
        vals.append(m)
        x = jnp.where(iota == first, neg_inf, x)
    top = jnp.concatenate(vals, axis=1)
    feat = feat_ref[...]
    av = jnp.concatenate([feat[:, :N_FEAT], top], axis=1)
    m = jnp.max(av, axis=1, keepdims=True)
    e = jnp.exp(av - m)
    s = jnp.sum(e, axis=1, keepdims=True)
    out_ref[...] = e / s


def kernel(item_scores, feat_scores, cand_item):
    del cand_item
    item = jnp.pad(item_scores, ((0, 0), (0, V_PAD - V)),
                   constant_values=-jnp.inf)
    feat = jnp.pad(feat_scores.astype(jnp.float32), ((0, 0), (0, 32 - N_FEAT)),
                   constant_values=-jnp.inf)
    grid = B // ROWS_PER_BLOCK
    zero = lambda i: (i, jnp.int32(0))
    out = pl.pallas_call(
        _topk_softmax_kernel,
        grid=(grid,),
        in_specs=[
            pl.BlockSpec((ROWS_PER_BLOCK, V_PAD), zero),
            pl.BlockSpec((ROWS_PER_BLOCK, 32), zero),
        ],
        out_specs=pl.BlockSpec((ROWS_PER_BLOCK, N_FEAT + TOPK), zero),
        out_shape=jax.ShapeDtypeStruct((B, N_FEAT + TOPK), jnp.float32),
    )(item, feat)
    return out
